# Initial kernel scaffold; baseline (speedup 1.0000x reference)
import jax
import jax.numpy as jnp
from jax.experimental import pallas as pl


def kernel(input_, weight):
    raise NotImplementedError("write your pallas kernel here")


# speedup vs baseline: 1.5861x; 1.5861x over previous
import functools

import jax
import jax.numpy as jnp
from jax import lax
from jax.experimental import pallas as pl
from jax.experimental.pallas import tpu as pltpu
from jax.experimental.pallas import tpu_sc as plsc

NC = 2
NS = 16
NW = NC * NS

D = 64
R = 128


@functools.partial(jax.jit, static_argnums=(2,))
def _gather(weight, idx, B):
    b_per_w = B // NW
    n_chunks = b_per_w // R
    mesh = plsc.VectorSubcoreMesh(
        core_axis_name="c", subcore_axis_name="s", num_cores=NC, num_subcores=NS
    )

    @functools.partial(
        pl.kernel,
        out_type=jax.ShapeDtypeStruct((B, D), jnp.float32),
        mesh=mesh,
        scratch_types=[
            pltpu.VMEM((R,), jnp.int32),
            pltpu.VMEM((R, D), jnp.float32),
            pltpu.SemaphoreType.DMA,
        ],
        compiler_params=pltpu.CompilerParams(use_tc_tiling_on_sc=False),
    )
    def body(weight_hbm, idx_hbm, out_hbm, idx_v, rows_v, sem):
        wid = lax.axis_index("s") * NC + lax.axis_index("c")
        base = wid * b_per_w

        @pl.loop(0, n_chunks)
        def _(j):
            cbase = base + j * R
            pltpu.sync_copy(idx_hbm.at[pl.ds(cbase, R)], idx_v)
            pltpu.async_copy(weight_hbm.at[idx_v], rows_v, sem).wait()
            pltpu.sync_copy(rows_v, out_hbm.at[pl.ds(cbase, R)])

    return body(weight, idx)


def kernel(input_, weight):
    Bm, S = input_.shape
    B = Bm * S
    idx = input_.reshape(B).astype(jnp.int32)
    out = _gather(weight, idx, B)
    return out.reshape(Bm, S, D)


# speedup vs baseline: 1.8795x; 1.1850x over previous
import functools

import jax
import jax.numpy as jnp
from jax import lax
from jax.experimental import pallas as pl
from jax.experimental.pallas import tpu as pltpu
from jax.experimental.pallas import tpu_sc as plsc

NC = 2
NS = 16
NW = NC * NS

D = 64
C = 128
NBUF = 4


@functools.partial(jax.jit, static_argnums=(2,))
def _gather(weight, idx, B):
    b_per_w = B // NW
    n_chunks = b_per_w // C
    mesh = plsc.VectorSubcoreMesh(
        core_axis_name="c", subcore_axis_name="s", num_cores=NC, num_subcores=NS
    )

    @functools.partial(
        pl.kernel,
        out_type=jax.ShapeDtypeStruct((B, D), jnp.float32),
        mesh=mesh,
        scratch_types=[
            pltpu.VMEM((n_chunks, C), jnp.int32),
            pltpu.VMEM((NBUF, C, D), jnp.float32),
            pltpu.SemaphoreType.DMA((NBUF,)),
            pltpu.SemaphoreType.DMA((NBUF,)),
        ],
        compiler_params=pltpu.CompilerParams(use_tc_tiling_on_sc=False),
    )
    def body(weight_hbm, idx_hbm, out_hbm, idx_v, bufs, gsem, wsem):
        wid = lax.axis_index("s") * NC + lax.axis_index("c")
        base = wid * b_per_w

        def gather_start(j, b):
            pltpu.async_copy(weight_hbm.at[idx_v.at[j]], bufs.at[b], gsem.at[b])

        def gather_wait(j, b):
            pltpu.make_async_copy(
                weight_hbm.at[idx_v.at[j]], bufs.at[b], gsem.at[b]
            ).wait()

        def wb_start(j, b):
            pltpu.async_copy(
                bufs.at[b], out_hbm.at[pl.ds(base + j * C, C)], wsem.at[b]
            )

        def wb_wait(j, b):
            pltpu.make_async_copy(
                bufs.at[b], out_hbm.at[pl.ds(base + j * C, C)], wsem.at[b]
            ).wait()

        pltpu.sync_copy(idx_hbm.at[wid], idx_v)
        for b in range(NBUF):
            gather_start(b, b)

        @pl.loop(0, n_chunks - NBUF, step=NBUF)
        def _(j0):
            for b in range(NBUF):
                j = j0 + b
                gather_wait(j, b)
                wb_start(j, b)
                wb_wait(j, b)
                gather_start(j + NBUF, b)

        for b in range(NBUF):
            jlast = n_chunks - NBUF + b
            gather_wait(jlast, b)
            wb_start(jlast, b)
        for b in range(NBUF):
            jlast = n_chunks - NBUF + b
            wb_wait(jlast, b)

    idx3 = idx.reshape(NW, n_chunks, C)
    return body(weight, idx3)


def kernel(input_, weight):
    Bm, S = input_.shape
    B = Bm * S
    idx = input_.reshape(B).astype(jnp.int32)
    out = _gather(weight, idx, B)
    return out.reshape(Bm, S, D)
